# Initial kernel scaffold; baseline (speedup 1.0000x reference)
import jax
import jax.numpy as jnp
from jax.experimental import pallas as pl


def kernel(x, gate_w, W1, W2, W3):
    raise NotImplementedError("write your pallas kernel here")


# speedup vs baseline: 1.7147x; 1.7147x over previous
import functools

import jax
import jax.numpy as jnp
from jax import lax
from jax.experimental import pallas as pl
from jax.experimental.pallas import tpu as pltpu

HIDDEN = 1024
FFN = 4096
E = 8
TOPK = 2

BLK = 256
FBLK = 1024
RBLK = 512


def _routing_kernel(x_ref, gw_ref, rw_ref, ids_ref):
    x = x_ref[...]
    logits = lax.dot_general(x, gw_ref[...], (((1,), (1,)), ((), ())),
                             preferred_element_type=jnp.float32)
    m = jnp.max(logits, axis=-1, keepdims=True)
    p = jnp.exp(logits - m)
    p = p / jnp.sum(p, axis=-1, keepdims=True)
    v1 = p[:, 0:1]
    i1 = jnp.zeros_like(v1, dtype=jnp.int32)
    for e in range(1, E):
        better = p[:, e:e + 1] > v1
        v1 = jnp.where(better, p[:, e:e + 1], v1)
        i1 = jnp.where(better, e, i1)
    neg = jnp.float32(-1.0)
    v2 = jnp.where(i1 == 0, neg, p[:, 0:1])
    i2 = jnp.zeros_like(i1)
    for e in range(1, E):
        cand = jnp.where(i1 == e, neg, p[:, e:e + 1])
        better = cand > v2
        v2 = jnp.where(better, cand, v2)
        i2 = jnp.where(better, e, i2)
    s = v1 + v2
    rw_ref[...] = jnp.concatenate([v1 / s, v2 / s], axis=1)
    ids_ref[...] = jnp.concatenate([i1, i2], axis=1)


def _routing(xt, gate_w):
    t = xt.shape[0]
    grid = (t // RBLK,)
    rw, ids = pl.pallas_call(
        _routing_kernel,
        grid=grid,
        in_specs=[
            pl.BlockSpec((RBLK, HIDDEN), lambda i: (i, 0)),
            pl.BlockSpec((E, HIDDEN), lambda i: (0, 0)),
        ],
        out_specs=[
            pl.BlockSpec((RBLK, TOPK), lambda i: (i, 0)),
            pl.BlockSpec((RBLK, TOPK), lambda i: (i, 0)),
        ],
        out_shape=[
            jax.ShapeDtypeStruct((t, TOPK), jnp.float32),
            jax.ShapeDtypeStruct((t, TOPK), jnp.int32),
        ],
    )(xt, gate_w)
    return rw, ids


def _dispatch_plan(ids, rw, t):
    n = t * TOPK
    nb = n // BLK + E
    p_rows = nb * BLK
    flat_e = ids.reshape(-1)
    onehot = (flat_e[:, None] == jnp.arange(E)[None, :]).astype(jnp.int32)
    ranks = jnp.cumsum(onehot, axis=0)
    counts = ranks[-1]
    rank_in_e = jnp.sum((ranks - 1) * onehot, axis=1)
    padded = ((counts + BLK - 1) // BLK) * BLK
    offs = jnp.concatenate([jnp.zeros((1,), jnp.int32),
                            jnp.cumsum(padded)[:-1].astype(jnp.int32)])
    pos = offs[flat_e] + rank_in_e
    flat_t = jnp.arange(n, dtype=jnp.int32) // TOPK
    sorted_tok = jnp.zeros((p_rows,), jnp.int32).at[pos].set(flat_t)
    sorted_w = jnp.zeros((p_rows,), jnp.float32).at[pos].set(rw.reshape(-1))
    end_blk = jnp.cumsum(padded // BLK)
    blk_ids = jnp.arange(nb, dtype=jnp.int32)
    block_expert = jnp.sum(blk_ids[:, None] >= end_blk[None, :],
                           axis=1).astype(jnp.int32)
    block_expert = jnp.minimum(block_expert, E - 1)
    return sorted_tok, sorted_w, pos.reshape(t, TOPK), block_expert, nb


def _ffn_a_kernel(be_ref, xs_ref, w1_ref, w3_ref, h_ref):
    del be_ref
    x = xs_ref[...].astype(jnp.bfloat16)
    w1 = w1_ref[0].astype(jnp.bfloat16)
    w3 = w3_ref[0].astype(jnp.bfloat16)
    a = lax.dot_general(x, w1, (((1,), (1,)), ((), ())),
                        preferred_element_type=jnp.float32)
    b = lax.dot_general(x, w3, (((1,), (1,)), ((), ())),
                        preferred_element_type=jnp.float32)
    h = (a * jax.nn.sigmoid(a)) * b
    h_ref[...] = h.astype(jnp.bfloat16)


def _ffn_b_kernel(be_ref, h_ref, w2_ref, sw_ref, y_ref):
    del be_ref
    h = h_ref[...]
    w2 = w2_ref[0].astype(jnp.bfloat16)
    y = lax.dot_general(h, w2, (((1,), (1,)), ((), ())),
                        preferred_element_type=jnp.float32)
    y_ref[...] = y * sw_ref[...]


def _grouped_ffn(xs, W1, W2, W3, sorted_w, block_expert, nb):
    p_rows = xs.shape[0]
    nf = FFN // FBLK
    h = pl.pallas_call(
        _ffn_a_kernel,
        grid_spec=pltpu.PrefetchScalarGridSpec(
            num_scalar_prefetch=1,
            grid=(nf, nb),
            in_specs=[
                pl.BlockSpec((BLK, HIDDEN), lambda f, b, be: (b, 0)),
                pl.BlockSpec((1, FBLK, HIDDEN), lambda f, b, be: (be[b], f, 0)),
                pl.BlockSpec((1, FBLK, HIDDEN), lambda f, b, be: (be[b], f, 0)),
            ],
            out_specs=pl.BlockSpec((BLK, FBLK), lambda f, b, be: (b, f)),
        ),
        out_shape=jax.ShapeDtypeStruct((p_rows, FFN), jnp.bfloat16),
        compiler_params=pltpu.CompilerParams(
            dimension_semantics=("arbitrary", "arbitrary")),
    )(block_expert, xs, W1, W3)
    y = pl.pallas_call(
        _ffn_b_kernel,
        grid_spec=pltpu.PrefetchScalarGridSpec(
            num_scalar_prefetch=1,
            grid=(nb,),
            in_specs=[
                pl.BlockSpec((BLK, FFN), lambda b, be: (b, 0)),
                pl.BlockSpec((1, HIDDEN, FFN), lambda b, be: (be[b], 0, 0)),
                pl.BlockSpec((BLK, 1), lambda b, be: (b, 0)),
            ],
            out_specs=pl.BlockSpec((BLK, HIDDEN), lambda b, be: (b, 0)),
        ),
        out_shape=jax.ShapeDtypeStruct((p_rows, HIDDEN), jnp.float32),
        compiler_params=pltpu.CompilerParams(
            dimension_semantics=("arbitrary",)),
    )(block_expert, h, W2, sorted_w.reshape(p_rows, 1))
    return y


def kernel(x, gate_w, W1, W2, W3):
    bs, sq, dim = x.shape
    t = bs * sq
    xt = x.reshape(t, dim)
    rw, ids = _routing(xt, gate_w)
    sorted_tok, sorted_w, pos, block_expert, nb = _dispatch_plan(ids, rw, t)
    xs = jnp.take(xt, sorted_tok, axis=0)
    y = _grouped_ffn(xs, W1, W2, W3, sorted_w, block_expert, nb)
    final = y[pos[:, 0]] + y[pos[:, 1]]
    return final.reshape(bs, sq, dim), rw


# speedup vs baseline: 1.8105x; 1.0559x over previous
import functools

import jax
import jax.numpy as jnp
from jax import lax
from jax.experimental import pallas as pl
from jax.experimental.pallas import tpu as pltpu

HIDDEN = 1024
FFN = 4096
E = 8
TOPK = 2

BLK = 256
FBLK = 1024
RBLK = 512


def _routing_kernel(x_ref, gw_ref, rw_ref, ids_ref):
    x = x_ref[...]
    logits = lax.dot_general(x, gw_ref[...], (((1,), (1,)), ((), ())),
                             preferred_element_type=jnp.float32)
    m = jnp.max(logits, axis=-1, keepdims=True)
    p = jnp.exp(logits - m)
    p = p / jnp.sum(p, axis=-1, keepdims=True)
    v1 = p[:, 0:1]
    i1 = jnp.zeros_like(v1, dtype=jnp.int32)
    for e in range(1, E):
        better = p[:, e:e + 1] > v1
        v1 = jnp.where(better, p[:, e:e + 1], v1)
        i1 = jnp.where(better, e, i1)
    neg = jnp.float32(-1.0)
    v2 = jnp.where(i1 == 0, neg, p[:, 0:1])
    i2 = jnp.zeros_like(i1)
    for e in range(1, E):
        cand = jnp.where(i1 == e, neg, p[:, e:e + 1])
        better = cand > v2
        v2 = jnp.where(better, cand, v2)
        i2 = jnp.where(better, e, i2)
    s = v1 + v2
    rw_ref[...] = jnp.concatenate([v1 / s, v2 / s], axis=1)
    ids_ref[...] = jnp.concatenate([i1, i2], axis=1)


def _routing(xt, gate_w):
    t = xt.shape[0]
    grid = (t // RBLK,)
    rw, ids = pl.pallas_call(
        _routing_kernel,
        grid=grid,
        in_specs=[
            pl.BlockSpec((RBLK, HIDDEN), lambda i: (i, 0)),
            pl.BlockSpec((E, HIDDEN), lambda i: (0, 0)),
        ],
        out_specs=[
            pl.BlockSpec((RBLK, TOPK), lambda i: (i, 0)),
            pl.BlockSpec((RBLK, TOPK), lambda i: (i, 0)),
        ],
        out_shape=[
            jax.ShapeDtypeStruct((t, TOPK), jnp.float32),
            jax.ShapeDtypeStruct((t, TOPK), jnp.int32),
        ],
    )(xt, gate_w)
    return rw, ids


def _dispatch_plan(ids, t):
    n = t * TOPK
    nb = n // BLK + E
    p_rows = nb * BLK
    flat_e = ids.reshape(-1)
    onehot = (flat_e[:, None] == jnp.arange(E)[None, :]).astype(jnp.int32)
    ranks = jnp.cumsum(onehot, axis=0)
    counts = ranks[-1]
    rank_in_e = jnp.sum((ranks - 1) * onehot, axis=1)
    padded = ((counts + BLK - 1) // BLK) * BLK
    offs = jnp.concatenate([jnp.zeros((1,), jnp.int32),
                            jnp.cumsum(padded)[:-1].astype(jnp.int32)])
    pos = offs[flat_e] + rank_in_e
    flat_t = jnp.arange(n, dtype=jnp.int32) // TOPK
    sorted_tok = jnp.zeros((p_rows,), jnp.int32).at[pos].set(flat_t)
    end_blk = jnp.cumsum(padded // BLK)
    blk_ids = jnp.arange(nb, dtype=jnp.int32)
    block_expert = jnp.sum(blk_ids[:, None] >= end_blk[None, :],
                           axis=1).astype(jnp.int32)
    block_expert = jnp.minimum(block_expert, E - 1)
    return sorted_tok, pos.reshape(t, TOPK), block_expert, nb


def _ffn_a_kernel(be_ref, xs_ref, w1_ref, w3_ref, h_ref):
    del be_ref
    x = xs_ref[...].astype(jnp.bfloat16)
    w1 = w1_ref[0].astype(jnp.bfloat16)
    w3 = w3_ref[0].astype(jnp.bfloat16)
    a = lax.dot_general(x, w1, (((1,), (1,)), ((), ())),
                        preferred_element_type=jnp.float32)
    b = lax.dot_general(x, w3, (((1,), (1,)), ((), ())),
                        preferred_element_type=jnp.float32)
    h = (a * jax.nn.sigmoid(a)) * b
    h_ref[...] = h.astype(jnp.bfloat16)


def _ffn_b_kernel(be_ref, h_ref, w2_ref, y_ref):
    del be_ref
    h = h_ref[...]
    w2 = w2_ref[0].astype(jnp.bfloat16)
    y_ref[...] = lax.dot_general(h, w2, (((1,), (1,)), ((), ())),
                                 preferred_element_type=jnp.float32)


def _grouped_ffn(xs, W1, W2, W3, block_expert, nb):
    p_rows = xs.shape[0]
    nf = FFN // FBLK
    h = pl.pallas_call(
        _ffn_a_kernel,
        grid_spec=pltpu.PrefetchScalarGridSpec(
            num_scalar_prefetch=1,
            grid=(nf, nb),
            in_specs=[
                pl.BlockSpec((BLK, HIDDEN), lambda f, b, be: (b, 0)),
                pl.BlockSpec((1, FBLK, HIDDEN), lambda f, b, be: (be[b], f, 0)),
                pl.BlockSpec((1, FBLK, HIDDEN), lambda f, b, be: (be[b], f, 0)),
            ],
            out_specs=pl.BlockSpec((BLK, FBLK), lambda f, b, be: (b, f)),
        ),
        out_shape=jax.ShapeDtypeStruct((p_rows, FFN), jnp.bfloat16),
        compiler_params=pltpu.CompilerParams(
            dimension_semantics=("arbitrary", "arbitrary")),
    )(block_expert, xs, W1, W3)
    y = pl.pallas_call(
        _ffn_b_kernel,
        grid_spec=pltpu.PrefetchScalarGridSpec(
            num_scalar_prefetch=1,
            grid=(nb,),
            in_specs=[
                pl.BlockSpec((BLK, FFN), lambda b, be: (b, 0)),
                pl.BlockSpec((1, HIDDEN, FFN), lambda b, be: (be[b], 0, 0)),
            ],
            out_specs=pl.BlockSpec((BLK, HIDDEN), lambda b, be: (b, 0)),
        ),
        out_shape=jax.ShapeDtypeStruct((p_rows, HIDDEN), jnp.float32),
        compiler_params=pltpu.CompilerParams(
            dimension_semantics=("arbitrary",)),
    )(block_expert, h, W2)
    return y


def kernel(x, gate_w, W1, W2, W3):
    bs, sq, dim = x.shape
    t = bs * sq
    xt = x.reshape(t, dim)
    rw, ids = _routing(xt, gate_w)
    sorted_tok, pos, block_expert, nb = _dispatch_plan(ids, t)
    xs = jnp.take(xt, sorted_tok, axis=0)
    y = _grouped_ffn(xs, W1, W2, W3, block_expert, nb)
    final = rw[:, 0:1] * y[pos[:, 0]] + rw[:, 1:2] * y[pos[:, 1]]
    return final.reshape(bs, sq, dim), rw
